# speedup vs baseline: 1.2952x; 1.2952x over previous
import functools

import jax
import jax.numpy as jnp
from jax import lax
from jax.experimental import pallas as pl
from jax.experimental.pallas import tpu as pltpu
from jax.experimental.pallas import tpu_sc as plsc

N = 16384
D = 512
L = 16
NC = 2
NS = 16
NW = NC * NS
ROWS_PER_W = N // NW
CHUNK = 64
N_CHUNKS = ROWS_PER_W // CHUNK


def _sie_body(feat_hbm, cam_ids_hbm, view_ids_hbm, cam_w_hbm, view_w_hbm,
              out_hbm, cam_idx_v, view_idx_v, acc_v, cam_v, view_v,
              sem_f, sem_c, sem_v):
    wid = lax.axis_index("s") * NC + lax.axis_index("c")
    base = wid * ROWS_PER_W

    pltpu.sync_copy(cam_ids_hbm.at[pl.ds(base, ROWS_PER_W)], cam_idx_v)
    pltpu.sync_copy(view_ids_hbm.at[pl.ds(base, ROWS_PER_W)], view_idx_v)

    def chunk_body(ci, carry):
        off = ci * CHUNK
        cf = pltpu.async_copy(feat_hbm.at[pl.ds(base + off, CHUNK)], acc_v,
                              sem_f)
        cc = pltpu.async_copy(cam_w_hbm.at[cam_idx_v.at[pl.ds(off, CHUNK)]],
                              cam_v, sem_c)
        cv = pltpu.async_copy(view_w_hbm.at[view_idx_v.at[pl.ds(off, CHUNK)]],
                              view_v, sem_v)
        cf.wait()
        cc.wait()
        cv.wait()

        def row_body(r, rcarry):
            for d in range(D // L):
                sl = pl.ds(d * L, L)
                acc_v[r, sl] = acc_v[r, sl] + cam_v[r, sl] + view_v[r, sl]
            return rcarry

        lax.fori_loop(0, CHUNK, row_body, 0)
        pltpu.sync_copy(acc_v, out_hbm.at[pl.ds(base + off, CHUNK)])
        return carry

    lax.fori_loop(0, N_CHUNKS, chunk_body, 0)


@jax.jit
def kernel(feat, cam_ids, view_ids, cam_weight, view_weight):
    mesh = plsc.VectorSubcoreMesh(core_axis_name="c", subcore_axis_name="s")
    sie = functools.partial(
        pl.kernel,
        mesh=mesh,
        out_type=jax.ShapeDtypeStruct((N, D), jnp.float32),
        scratch_types=[
            pltpu.VMEM((ROWS_PER_W,), jnp.int32),
            pltpu.VMEM((ROWS_PER_W,), jnp.int32),
            pltpu.VMEM((CHUNK, D), jnp.float32),
            pltpu.VMEM((CHUNK, D), jnp.float32),
            pltpu.VMEM((CHUNK, D), jnp.float32),
            pltpu.SemaphoreType.DMA,
            pltpu.SemaphoreType.DMA,
            pltpu.SemaphoreType.DMA,
        ],
    )(_sie_body)
    return sie(feat, cam_ids.astype(jnp.int32), view_ids.astype(jnp.int32),
               cam_weight, view_weight)


# speedup vs baseline: 1.7976x; 1.3878x over previous
import functools

import jax
import jax.numpy as jnp
from jax import lax
from jax.experimental import pallas as pl
from jax.experimental.pallas import tpu as pltpu
from jax.experimental.pallas import tpu_sc as plsc

N = 16384
D = 512
L = 16
NC = 2
NS = 16
NW = NC * NS
ROWS_PER_W = N // NW
CHUNK = 16
N_CHUNKS = ROWS_PER_W // CHUNK
NBUF = 2


def _sie_body(feat_hbm, cam_ids_hbm, view_ids_hbm, cam_w_hbm, view_w_hbm,
              out_hbm, cam_idx_v, view_idx_v,
              f0, c0, v0, o0, f1, c1, v1, o1,
              gs0, gs1, ss0, ss1):
    wid = lax.axis_index("s") * NC + lax.axis_index("c")
    base = wid * ROWS_PER_W

    feat_bufs = (f0, f1)
    cam_bufs = (c0, c1)
    view_bufs = (v0, v1)
    out_bufs = (o0, o1)
    gsems = (gs0, gs1)
    ssems = (ss0, ss1)

    pltpu.sync_copy(cam_ids_hbm.at[pl.ds(base, ROWS_PER_W)], cam_idx_v)
    pltpu.sync_copy(view_ids_hbm.at[pl.ds(base, ROWS_PER_W)], view_idx_v)

    def gissue(c, b):
        off = c * CHUNK
        pltpu.async_copy(feat_hbm.at[pl.ds(base + off, CHUNK)],
                         feat_bufs[b], gsems[b])
        pltpu.async_copy(cam_w_hbm.at[cam_idx_v.at[pl.ds(off, CHUNK)]],
                         cam_bufs[b], gsems[b])
        pltpu.async_copy(view_w_hbm.at[view_idx_v.at[pl.ds(off, CHUNK)]],
                         view_bufs[b], gsems[b])

    def gwait(b):
        for dst in (feat_bufs[b], cam_bufs[b], view_bufs[b]):
            pltpu.make_async_copy(feat_hbm.at[pl.ds(0, CHUNK)], dst,
                                  gsems[b]).wait()

    def swait(b):
        pltpu.make_async_copy(out_bufs[b], out_hbm.at[pl.ds(0, CHUNK)],
                              ssems[b]).wait()

    gissue(0, 0)
    gissue(1, 1)

    def pair_body(j, carry):
        for b in range(NBUF):
            c = j * NBUF + b
            fb, cb, vb, ob = feat_bufs[b], cam_bufs[b], view_bufs[b], out_bufs[b]
            gwait(b)

            @pl.when(c >= NBUF)
            def _():
                swait(b)

            def row_body(r, rcarry):
                for d in range(D // L):
                    sl = pl.ds(d * L, L)
                    ob[r, sl] = fb[r, sl] + cb[r, sl] + vb[r, sl]
                return rcarry

            lax.fori_loop(0, CHUNK, row_body, 0)
            pltpu.async_copy(ob, out_hbm.at[pl.ds(base + c * CHUNK, CHUNK)],
                             ssems[b])

            @pl.when(c + NBUF < N_CHUNKS)
            def _():
                gissue(c + NBUF, b)
        return carry

    lax.fori_loop(0, N_CHUNKS // NBUF, pair_body, 0)

    swait(0)
    swait(1)


@jax.jit
def kernel(feat, cam_ids, view_ids, cam_weight, view_weight):
    mesh = plsc.VectorSubcoreMesh(core_axis_name="c", subcore_axis_name="s")
    buf = pltpu.VMEM((CHUNK, D), jnp.float32)
    sie = functools.partial(
        pl.kernel,
        mesh=mesh,
        out_type=jax.ShapeDtypeStruct((N, D), jnp.float32),
        scratch_types=[
            pltpu.VMEM((ROWS_PER_W,), jnp.int32),
            pltpu.VMEM((ROWS_PER_W,), jnp.int32),
            buf, buf, buf, buf,
            buf, buf, buf, buf,
            pltpu.SemaphoreType.DMA,
            pltpu.SemaphoreType.DMA,
            pltpu.SemaphoreType.DMA,
            pltpu.SemaphoreType.DMA,
        ],
    )(_sie_body)
    return sie(feat, cam_ids.astype(jnp.int32), view_ids.astype(jnp.int32),
               cam_weight, view_weight)
